# Initial kernel scaffold; baseline (speedup 1.0000x reference)
import jax
import jax.numpy as jnp
from jax.experimental import pallas as pl


def kernel(predDepth, invcamK, semanticLabel, pixelLocs, bind, lind, bias_helper, permute_index):
    raise NotImplementedError("write your pallas kernel here")


# speedup vs baseline: 63.3176x; 63.3176x over previous
import functools

import jax
import jax.numpy as jnp
from jax import lax
from jax.experimental import pallas as pl
from jax.experimental.pallas import tpu as pltpu
from jax.experimental.pallas import tpu_sc as plsc

B_, H_, W_, P_ = 4, 320, 1024, 10000
HW = H_ * W_
MAXD = 40.0
TYPEIND = 5
NC, NS, LANES = 2, 16, 16
NW = NC * NS
SEG = NW // B_
JCH = HW // SEG
WORDS = HW // 16
ROWS = B_ * HW // W_
RB = 64
PCH = 1264
NIT_HI, NIT_LO = 79, 72


def _maskpack_body(sem_ref, dep_ref, p2_ref, out_ref):
    m = jnp.logical_and(sem_ref[...] == TYPEIND, dep_ref[...] < MAXD)
    w = lax.dot_general(m.astype(jnp.float32), p2_ref[...],
                        (((1,), (0,)), ((), ())),
                        preferred_element_type=jnp.float32)
    out_ref[...] = w.astype(jnp.int32)


_maskpack = pl.pallas_call(
    _maskpack_body,
    grid=(ROWS // RB,),
    in_specs=[
        pl.BlockSpec((RB, 1024), lambda i: (i, 0)),
        pl.BlockSpec((RB, 1024), lambda i: (i, 0)),
        pl.BlockSpec((1024, 64), lambda i: (0, 0)),
    ],
    out_specs=pl.BlockSpec((RB, 64), lambda i: (i, 0)),
    out_shape=jax.ShapeDtypeStruct((ROWS, 64), jnp.int32),
)


_sc_mesh = plsc.VectorSubcoreMesh(
    core_axis_name="c", subcore_axis_name="s", num_cores=NC, num_subcores=NS)


@functools.partial(
    pl.kernel,
    out_type=(
        jax.ShapeDtypeStruct((NW * JCH,), jnp.int32),
        jax.ShapeDtypeStruct((NW * 16,), jnp.int32),
    ),
    mesh=_sc_mesh,
    compiler_params=pltpu.CompilerParams(needs_layout_passes=False),
    scratch_types=[
        pltpu.VMEM((JCH,), jnp.int32),
        pltpu.VMEM((WORDS,), jnp.int32),
        pltpu.VMEM((JCH + 16,), jnp.int32),
        pltpu.VMEM((16,), jnp.int32),
    ],
)
def _compact(perm_hbm, tbl_hbm, lists_hbm, counts_hbm,
             perm_v, tbl_v, list_v, cnt_v):
    c = lax.axis_index("c")
    s = lax.axis_index("s")
    wid = c * NS + s
    batch = wid // SEG
    seg = wid % SEG
    pltpu.sync_copy(perm_hbm.at[pl.ds(seg * JCH, JCH)], perm_v)
    pltpu.sync_copy(tbl_hbm.at[pl.ds(batch * WORDS, WORDS)], tbl_v)

    def body(i, cnt):
        idx = perm_v[pl.ds(i * 16, 16)]
        wv = plsc.load_gather(tbl_v, [lax.shift_right_logical(idx, 4)])
        bit = lax.shift_right_logical(wv, idx & 15) & 1
        plsc.store_compressed(list_v.at[pl.ds(cnt, 16)], idx, mask=bit > 0)
        return cnt + jnp.sum(bit)

    cnt = lax.fori_loop(0, JCH // 16, body, jnp.int32(0))
    cnt_v[...] = jnp.full((16,), cnt, jnp.int32)
    pltpu.sync_copy(cnt_v, counts_hbm.at[pl.ds(wid * 16, 16)])
    pltpu.sync_copy(list_v.at[pl.ds(0, JCH)], lists_hbm.at[pl.ds(wid * JCH, JCH)])


@functools.partial(
    pl.kernel,
    out_type=jax.ShapeDtypeStruct((B_ * 3 * P_,), jnp.float32),
    mesh=_sc_mesh,
    compiler_params=pltpu.CompilerParams(needs_layout_passes=False),
    scratch_types=[
        pltpu.VMEM((NW * 16,), jnp.int32),
        pltpu.VMEM((P_,), jnp.int32),
        pltpu.VMEM((PCH + 16,), jnp.int32),
        pltpu.VMEM((PCH + 16,), jnp.int32),
        pltpu.VMEM((PCH + 16,), jnp.float32),
        pltpu.VMEM((64,), jnp.float32),
        pltpu.VMEM((3 * PCH,), jnp.float32),
        pltpu.SemaphoreType.DMA,
    ],
)
def _sample(counts_hbm, bind_hbm, lists_hbm, depth_hbm, k_hbm, out_hbm,
            cnts_v, bind_v, idx_v, lval_v, dval_v, k_v, row_v, sem):
    c = lax.axis_index("c")
    s = lax.axis_index("s")
    wid = c * NS + s
    batch = wid // SEG
    pchunk = wid % SEG
    p0 = pchunk * PCH

    pltpu.sync_copy(counts_hbm, cnts_v)
    pltpu.sync_copy(bind_hbm.at[pl.ds(batch * P_, P_)], bind_v)
    pltpu.sync_copy(k_hbm, k_v)

    offs = []
    off = jnp.int32(0)
    for t in range(SEG):
        offs.append(off)
        off = off + cnts_v[pl.ds((batch * SEG + t) * 16, 16)][0]
    vn = off

    vnb = []
    for bb in range(B_):
        tot = jnp.int32(0)
        for t in range(SEG):
            tot = tot + cnts_v[pl.ds((bb * SEG + t) * 16, 16)][0]
        vnb.append(tot)
    b0 = jnp.int32(B_ - 1)
    for bb in range(B_ - 2, -1, -1):
        b0 = jnp.where(vnb[bb] > 0, jnp.int32(bb), b0)
    t0 = jnp.int32(SEG - 1)
    for t in range(SEG - 2, -1, -1):
        t0 = jnp.where(cnts_v[pl.ds((b0 * SEG + t) * 16, 16)][0] > 0, jnp.int32(t), t0)
    fb_row = (b0 * SEG + t0) * JCH
    total_valid = vnb[0] > 0
    for bb in range(1, B_):
        total_valid = jnp.logical_or(total_valid, vnb[bb] > 0)
    tv = jnp.where(total_valid, jnp.float32(1.0), jnp.float32(0.0))

    niters = jnp.where(pchunk < SEG - 1, jnp.int32(NIT_HI), jnp.int32(NIT_LO))
    nchunks = jnp.where(pchunk < SEG - 1, jnp.int32(10), jnp.int32(9))
    vn_eff = jnp.maximum(vn, 1)
    has_pts = vn > 0

    idx_v[pl.ds(NIT_LO * 16, 16)] = jnp.zeros((16,), jnp.int32)
    idx_v[pl.ds(NIT_HI * 16, 16)] = jnp.zeros((16,), jnp.int32)

    def pass1(i, _):
        bv = bind_v[pl.ds(p0 + i * 16, 16)]
        ii = bv % vn_eff
        tseg = jnp.zeros((16,), jnp.int32)
        for t in range(1, SEG):
            tseg = tseg + (ii >= offs[t]).astype(jnp.int32)
        obase = jnp.zeros((16,), jnp.int32)
        for t in range(1, SEG):
            obase = jnp.where(tseg == t, offs[t], obase)
        rowi = (batch * SEG + tseg) * JCH + (ii - obase)
        rowi = jnp.where(has_pts, rowi, jnp.full((16,), fb_row, jnp.int32))
        idx_v[pl.ds(i * 16, 16)] = rowi
        return _

    lax.fori_loop(0, niters, pass1, jnp.int32(0))

    def g1(k, _):
        pltpu.async_copy(lists_hbm.at[idx_v.at[pl.ds(k * 128, 128)]],
                         lval_v.at[pl.ds(k * 128, 128)], sem).wait()
        return _

    lax.fori_loop(0, nchunks, g1, jnp.int32(0))

    borig = jnp.where(has_pts, batch, b0)

    def pass3(i, _):
        lv = lval_v[pl.ds(i * 16, 16)]
        g = jnp.clip(borig * HW + lv, 0, B_ * HW - 1)
        idx_v[pl.ds(i * 16, 16)] = g
        return _

    lax.fori_loop(0, niters, pass3, jnp.int32(0))

    def g2(k, _):
        pltpu.async_copy(depth_hbm.at[idx_v.at[pl.ds(k * 128, 128)]],
                         dval_v.at[pl.ds(k * 128, 128)], sem).wait()
        return _

    lax.fori_loop(0, nchunks, g2, jnp.int32(0))

    def pass4(i, _):
        g = idx_v[pl.ds(i * 16, 16)]
        d = dval_v[pl.ds(i * 16, 16)]
        bv = ((g >= HW).astype(jnp.int32) + (g >= 2 * HW).astype(jnp.int32)
              + (g >= 3 * HW).astype(jnp.int32))
        r = g - bv * HW
        yv = lax.shift_right_logical(r, 10).astype(jnp.float32)
        xv = (r & 1023).astype(jnp.float32)
        xd = xv * d
        yd = yv * d
        kbase = bv * 16
        for crow in range(3):
            k0 = plsc.load_gather(k_v, [kbase + (crow * 4 + 0)])
            k1 = plsc.load_gather(k_v, [kbase + (crow * 4 + 1)])
            k2 = plsc.load_gather(k_v, [kbase + (crow * 4 + 2)])
            k3 = plsc.load_gather(k_v, [kbase + (crow * 4 + 3)])
            val = (k0 * xd + k1 * yd + k2 * d + k3) * tv
            row_v[pl.ds(crow * PCH + i * 16, 16)] = val
        return _

    lax.fori_loop(0, niters, pass4, jnp.int32(0))

    for crow in range(3):
        obase = batch * (3 * P_) + crow * P_ + p0
        pltpu.sync_copy(row_v.at[pl.ds(crow * PCH, NIT_LO * 16)],
                        out_hbm.at[pl.ds(obase, NIT_LO * 16)])

    @pl.when(pchunk < SEG - 1)
    def _tail():
        for crow in range(3):
            obase = batch * (3 * P_) + crow * P_ + p0 + NIT_LO * 16
            pltpu.sync_copy(row_v.at[pl.ds(crow * PCH + NIT_LO * 16, PCH - NIT_LO * 16)],
                            out_hbm.at[pl.ds(obase, PCH - NIT_LO * 16)])


def _pow2_matrix():
    j = jnp.arange(1024)
    col = j // 16
    bit = (j % 16).astype(jnp.int32)
    vals = jnp.left_shift(jnp.int32(1), bit).astype(jnp.float32)
    onehot = (col[:, None] == jnp.arange(64)[None, :]).astype(jnp.float32)
    return onehot * vals[:, None]


@jax.jit
def kernel(predDepth, invcamK, semanticLabel, pixelLocs, bind, lind,
           bias_helper, permute_index):
    sem2 = semanticLabel.reshape(ROWS, 1024)
    dep2 = predDepth.reshape(ROWS, 1024)
    tbl = _maskpack(sem2, dep2, _pow2_matrix()).reshape(-1)
    perm_i = permute_index.astype(jnp.int32)
    lists, counts = _compact(perm_i, tbl)
    bind_i = bind.astype(jnp.int32).reshape(-1)
    depth_f = predDepth.reshape(-1)
    k_f = invcamK.reshape(-1)
    out_flat = _sample(counts, bind_i, lists, depth_f, k_f)
    pts = out_flat.reshape(B_, 3, P_)
    vn = counts.reshape(NW, 16)[:, 0].reshape(B_, SEG).sum(axis=1)
    vbi = (vn > 0).astype(jnp.float32).reshape(B_, 1)
    return pts, vbi


# speedup vs baseline: 77.3845x; 1.2222x over previous
import functools

import jax
import jax.numpy as jnp
from jax import lax
from jax.experimental import pallas as pl
from jax.experimental.pallas import tpu as pltpu
from jax.experimental.pallas import tpu_sc as plsc

B_, H_, W_, P_ = 4, 320, 1024, 10000
HW = H_ * W_
MAXD = 40.0
TYPEIND = 5
NC, NS, LANES = 2, 16, 16
NW = NC * NS
SEG = NW // B_
JCH = HW // SEG
HALF = JCH // 2
HALFPAD = HALF + 16
SEGV = SEG * 2
WORDS = HW // 16
ROWS = B_ * HW // W_
RB = 64
PCH = 1264
NIT_HI, NIT_LO = 79, 72


def _maskpack_body(sem_ref, dep_ref, p2_ref, out_ref):
    m = jnp.logical_and(sem_ref[...] == TYPEIND, dep_ref[...] < MAXD)
    w = lax.dot_general(m.astype(jnp.float32), p2_ref[...],
                        (((1,), (0,)), ((), ())),
                        preferred_element_type=jnp.float32)
    out_ref[...] = w.astype(jnp.int32)


_maskpack = pl.pallas_call(
    _maskpack_body,
    grid=(ROWS // RB,),
    in_specs=[
        pl.BlockSpec((RB, 1024), lambda i: (i, 0)),
        pl.BlockSpec((RB, 1024), lambda i: (i, 0)),
        pl.BlockSpec((1024, 64), lambda i: (0, 0)),
    ],
    out_specs=pl.BlockSpec((RB, 64), lambda i: (i, 0)),
    out_shape=jax.ShapeDtypeStruct((ROWS, 64), jnp.int32),
)


_sc_mesh = plsc.VectorSubcoreMesh(
    core_axis_name="c", subcore_axis_name="s", num_cores=NC, num_subcores=NS)


@functools.partial(
    pl.kernel,
    out_type=(
        jax.ShapeDtypeStruct((NW * 2 * HALF,), jnp.int32),
        jax.ShapeDtypeStruct((NW * 2 * 16,), jnp.int32),
    ),
    mesh=_sc_mesh,
    compiler_params=pltpu.CompilerParams(needs_layout_passes=False),
    scratch_types=[
        pltpu.VMEM((JCH,), jnp.int32),
        pltpu.VMEM((WORDS,), jnp.int32),
        pltpu.VMEM((2 * HALFPAD,), jnp.int32),
        pltpu.VMEM((16,), jnp.int32),
    ],
)
def _compact(perm_hbm, tbl_hbm, lists_hbm, counts_hbm,
             perm_v, tbl_v, list_v, cnt_v):
    c = lax.axis_index("c")
    s = lax.axis_index("s")
    wid = c * NS + s
    batch = wid // SEG
    seg = wid % SEG
    pltpu.sync_copy(perm_hbm.at[pl.ds(seg * JCH, JCH)], perm_v)
    pltpu.sync_copy(tbl_hbm.at[pl.ds(batch * WORDS, WORDS)], tbl_v)

    def body(i, carry):
        cnt_a, cnt_b = carry
        for u in range(2):
            idx_a = perm_v[pl.ds(i * 32 + u * 16, 16)]
            idx_b = perm_v[pl.ds(HALF + i * 32 + u * 16, 16)]
            wv_a = plsc.load_gather(tbl_v, [lax.shift_right_logical(idx_a, 4)])
            wv_b = plsc.load_gather(tbl_v, [lax.shift_right_logical(idx_b, 4)])
            msk_a = (lax.shift_right_logical(wv_a, idx_a & 15) & 1) > 0
            msk_b = (lax.shift_right_logical(wv_b, idx_b & 15) & 1) > 0
            plsc.store_compressed(list_v.at[pl.ds(cnt_a, 16)], idx_a, mask=msk_a)
            plsc.store_compressed(list_v.at[pl.ds(HALFPAD + cnt_b, 16)], idx_b,
                                  mask=msk_b)
            cnt_a = cnt_a + plsc.all_reduce_population_count(msk_a)[0]
            cnt_b = cnt_b + plsc.all_reduce_population_count(msk_b)[0]
        return cnt_a, cnt_b

    cnt_a, cnt_b = lax.fori_loop(0, HALF // 32, body,
                                 (jnp.int32(0), jnp.int32(0)))
    cnt_v[...] = jnp.full((16,), cnt_a, jnp.int32)
    pltpu.sync_copy(cnt_v, counts_hbm.at[pl.ds(wid * 32, 16)])
    cnt_v[...] = jnp.full((16,), cnt_b, jnp.int32)
    pltpu.sync_copy(cnt_v, counts_hbm.at[pl.ds(wid * 32 + 16, 16)])
    pltpu.sync_copy(list_v.at[pl.ds(0, HALF)],
                    lists_hbm.at[pl.ds(wid * 2 * HALF, HALF)])
    pltpu.sync_copy(list_v.at[pl.ds(HALFPAD, HALF)],
                    lists_hbm.at[pl.ds((wid * 2 + 1) * HALF, HALF)])


@functools.partial(
    pl.kernel,
    out_type=jax.ShapeDtypeStruct((B_ * 3 * P_,), jnp.float32),
    mesh=_sc_mesh,
    compiler_params=pltpu.CompilerParams(needs_layout_passes=False),
    scratch_types=[
        pltpu.VMEM((NW * 2 * 16,), jnp.int32),
        pltpu.VMEM((16,), jnp.int32),
        pltpu.VMEM((P_,), jnp.float32),
        pltpu.VMEM((PCH + 16,), jnp.int32),
        pltpu.VMEM((PCH + 16,), jnp.int32),
        pltpu.VMEM((PCH + 16,), jnp.float32),
        pltpu.VMEM((64,), jnp.float32),
        pltpu.VMEM((3 * PCH,), jnp.float32),
        pltpu.SemaphoreType.DMA,
    ],
)
def _sample(counts_hbm, bind_hbm, lists_hbm, depth_hbm, k_hbm, out_hbm,
            cnts_v, offs_v, bind_v, idx_v, lval_v, dval_v, k_v, row_v, sem):
    c = lax.axis_index("c")
    s = lax.axis_index("s")
    wid = c * NS + s
    batch = wid // SEG
    pchunk = wid % SEG
    p0 = pchunk * PCH

    pltpu.sync_copy(counts_hbm, cnts_v)
    pltpu.sync_copy(bind_hbm.at[pl.ds(batch * P_, P_)], bind_v)
    pltpu.sync_copy(k_hbm, k_v)

    lanes = lax.iota(jnp.int32, 16)
    cvec = plsc.load_gather(cnts_v, [lanes * 16 + batch * (SEGV * 16)])
    csum = plsc.cumsum(cvec)
    vn = csum[15]
    offs_v[...] = csum - cvec
    per_batch = []
    for bb in range(B_):
        cb = plsc.load_gather(cnts_v, [lanes * 16 + bb * (SEGV * 16)])
        per_batch.append(cb)
    vnb = [jnp.sum(cb) for cb in per_batch]
    b0 = jnp.int32(B_ - 1)
    for bb in range(B_ - 2, -1, -1):
        b0 = jnp.where(vnb[bb] > 0, jnp.int32(bb), b0)
    c0 = plsc.load_gather(cnts_v, [lanes * 16 + b0 * (SEGV * 16)])
    cum0 = plsc.cumsum(c0)
    t0 = plsc.all_reduce_population_count(cum0 == 0)[0]
    t0 = jnp.minimum(t0, jnp.int32(SEGV - 1))
    fb_row = (b0 * SEGV + t0) * HALF
    total_valid = vnb[0] > 0
    for bb in range(1, B_):
        total_valid = jnp.logical_or(total_valid, vnb[bb] > 0)
    tv = jnp.where(total_valid, jnp.float32(1.0), jnp.float32(0.0))

    niters = jnp.where(pchunk < SEG - 1, jnp.int32(NIT_HI), jnp.int32(NIT_LO))
    nchunks = jnp.where(pchunk < SEG - 1, jnp.int32(10), jnp.int32(9))
    vn_eff = jnp.maximum(vn, 1)
    has_pts = vn > 0

    idx_v[pl.ds(NIT_LO * 16, 16)] = jnp.zeros((16,), jnp.int32)
    idx_v[pl.ds(NIT_HI * 16, 16)] = jnp.zeros((16,), jnp.int32)

    def pass1(i, _):
        bv = bind_v[pl.ds(p0 + i * 16, 16)].astype(jnp.int32)
        ii = bv % vn_eff
        tseg = jnp.zeros((16,), jnp.int32)
        for step in (8, 4, 2, 1):
            cand = tseg + step
            oc = plsc.load_gather(offs_v, [cand])
            tseg = jnp.where(ii >= oc, cand, tseg)
        obase = plsc.load_gather(offs_v, [tseg])
        rowi = (batch * SEGV + tseg) * HALF + (ii - obase)
        rowi = jnp.where(has_pts, rowi, jnp.full((16,), fb_row, jnp.int32))
        idx_v[pl.ds(i * 16, 16)] = rowi
        return _

    lax.fori_loop(0, niters, pass1, jnp.int32(0))

    def g1(k, _):
        pltpu.async_copy(lists_hbm.at[idx_v.at[pl.ds(k * 128, 128)]],
                         lval_v.at[pl.ds(k * 128, 128)], sem)
        return _

    def g1_drain(k, _):
        pltpu.make_async_copy(lists_hbm.at[pl.ds(0, 128)],
                              lval_v.at[pl.ds(k * 128, 128)], sem).wait()
        return _

    lax.fori_loop(0, nchunks, g1, jnp.int32(0))
    lax.fori_loop(0, nchunks, g1_drain, jnp.int32(0))

    borig = jnp.where(has_pts, batch, b0)

    def pass3(i, _):
        lv = lval_v[pl.ds(i * 16, 16)]
        g = jnp.clip(borig * HW + lv, 0, B_ * HW - 1)
        idx_v[pl.ds(i * 16, 16)] = g
        return _

    lax.fori_loop(0, niters, pass3, jnp.int32(0))

    def g2(k, _):
        pltpu.async_copy(depth_hbm.at[idx_v.at[pl.ds(k * 128, 128)]],
                         dval_v.at[pl.ds(k * 128, 128)], sem)
        return _

    def g2_drain(k, _):
        pltpu.make_async_copy(depth_hbm.at[pl.ds(0, 128)],
                              dval_v.at[pl.ds(k * 128, 128)], sem).wait()
        return _

    lax.fori_loop(0, nchunks, g2, jnp.int32(0))
    lax.fori_loop(0, nchunks, g2_drain, jnp.int32(0))

    def pass4(i, _):
        g = idx_v[pl.ds(i * 16, 16)]
        d = dval_v[pl.ds(i * 16, 16)]
        bv = ((g >= HW).astype(jnp.int32) + (g >= 2 * HW).astype(jnp.int32)
              + (g >= 3 * HW).astype(jnp.int32))
        r = g - bv * HW
        yv = lax.shift_right_logical(r, 10).astype(jnp.float32)
        xv = (r & 1023).astype(jnp.float32)
        xd = xv * d
        yd = yv * d
        kbase = bv * 16
        for crow in range(3):
            k0 = plsc.load_gather(k_v, [kbase + (crow * 4 + 0)])
            k1 = plsc.load_gather(k_v, [kbase + (crow * 4 + 1)])
            k2 = plsc.load_gather(k_v, [kbase + (crow * 4 + 2)])
            k3 = plsc.load_gather(k_v, [kbase + (crow * 4 + 3)])
            val = (k0 * xd + k1 * yd + k2 * d + k3) * tv
            row_v[pl.ds(crow * PCH + i * 16, 16)] = val
        return _

    lax.fori_loop(0, niters, pass4, jnp.int32(0))

    for crow in range(3):
        obase = batch * (3 * P_) + crow * P_ + p0
        pltpu.sync_copy(row_v.at[pl.ds(crow * PCH, NIT_LO * 16)],
                        out_hbm.at[pl.ds(obase, NIT_LO * 16)])

    @pl.when(pchunk < SEG - 1)
    def _tail():
        for crow in range(3):
            obase = batch * (3 * P_) + crow * P_ + p0 + NIT_LO * 16
            pltpu.sync_copy(row_v.at[pl.ds(crow * PCH + NIT_LO * 16, PCH - NIT_LO * 16)],
                            out_hbm.at[pl.ds(obase, PCH - NIT_LO * 16)])


def _pow2_matrix():
    j = jnp.arange(1024)
    col = j // 16
    bit = (j % 16).astype(jnp.int32)
    vals = jnp.left_shift(jnp.int32(1), bit).astype(jnp.float32)
    onehot = (col[:, None] == jnp.arange(64)[None, :]).astype(jnp.float32)
    return onehot * vals[:, None]


@jax.jit
def kernel(predDepth, invcamK, semanticLabel, pixelLocs, bind, lind,
           bias_helper, permute_index):
    sem2 = semanticLabel.reshape(ROWS, 1024)
    dep2 = predDepth.reshape(ROWS, 1024)
    tbl = _maskpack(sem2, dep2, _pow2_matrix()).reshape(-1)
    perm_i = permute_index.astype(jnp.int32)
    lists, counts = _compact(perm_i, tbl)
    bind_f = bind.reshape(-1)
    depth_f = predDepth.reshape(-1)
    k_f = invcamK.reshape(-1)
    out_flat = _sample(counts, bind_f, lists, depth_f, k_f)
    pts = out_flat.reshape(B_, 3, P_)
    vn = counts.reshape(NW * 2, 16)[:, 0].reshape(B_, SEGV).sum(axis=1)
    vbi = (vn > 0).astype(jnp.float32).reshape(B_, 1)
    return pts, vbi
